# Initial kernel scaffold; baseline (speedup 1.0000x reference)
import jax
import jax.numpy as jnp
from jax.experimental import pallas as pl


def kernel(input, target, W_head, W1_0, W2_0, W1_1, W2_1):
    raise NotImplementedError("write your pallas kernel here")


# speedup vs baseline: 2.7412x; 2.7412x over previous
import functools

import jax
import jax.numpy as jnp
from jax.experimental import pallas as pl
from jax.experimental.pallas import tpu as pltpu

C0 = 2000
C1 = 20000
C2 = 100000
HEAD = 2002
HEAD_PAD = 2048
V0 = 18000
V1 = 80000
CHUNK = 2048
RB = 512


def _fused_kernel(x_ref, t_ref, wh_ref, w10_ref, w20_ref, w11_ref, w21_ref,
                  out_ref, *, rb):
    x = x_ref[...]
    t = t_ref[0]

    h0 = jax.lax.dot_general(x, w10_ref[...], (((1,), (1,)), ((), ())),
                             preferred_element_type=jnp.float32)
    h0 = h0.astype(jnp.bfloat16)
    h1 = jax.lax.dot_general(x, w11_ref[...], (((1,), (1,)), ((), ())),
                             preferred_element_type=jnp.float32)
    h1 = h1.astype(jnp.bfloat16)

    lg = jax.lax.dot_general(x, wh_ref[...], (((1,), (1,)), ((), ())),
                             preferred_element_type=jnp.float32)
    col = jax.lax.broadcasted_iota(jnp.int32, (rb, HEAD_PAD), 1)
    lg_m = jnp.where(col < HEAD, lg, -jnp.inf)
    m = jnp.max(lg_m, axis=1, keepdims=True)
    s = jnp.sum(jnp.exp(lg_m - m), axis=1, keepdims=True)
    lse_h = m + jnp.log(s)
    gidx = jnp.where(t < C0, t, jnp.where(t < C1, C0, C0 + 1))
    at_h = jnp.sum(jnp.where(col == gidx, lg, 0.0), axis=1, keepdims=True)
    head_term = at_h - lse_h

    def tail(h, w_ref, vocab, rel):
        nchunks = w_ref.shape[0] // CHUNK

        def body(c, carry):
            m, s, at = carry
            w = w_ref[pl.ds(c * CHUNK, CHUNK), :]
            lg = jax.lax.dot_general(h, w, (((1,), (1,)), ((), ())),
                                     preferred_element_type=jnp.float32)
            col = jax.lax.broadcasted_iota(jnp.int32, (rb, CHUNK), 1) \
                + c * CHUNK
            lg_m = jnp.where(col < vocab, lg, -jnp.inf)
            m_new = jnp.maximum(m, jnp.max(lg_m, axis=1, keepdims=True))
            s = s * jnp.exp(m - m_new) \
                + jnp.sum(jnp.exp(lg_m - m_new), axis=1, keepdims=True)
            at = at + jnp.sum(jnp.where(col == rel, lg, 0.0), axis=1,
                              keepdims=True)
            return m_new, s, at

        neg = jnp.full((rb, 1), -jnp.inf, dtype=jnp.float32)
        zero = jnp.zeros((rb, 1), dtype=jnp.float32)
        m, s, at = jax.lax.fori_loop(0, nchunks, body, (neg, zero, zero))
        return at - (m + jnp.log(s))

    local0 = tail(h0, w20_ref, V0, t - C0)
    local1 = tail(h1, w21_ref, V1, t - C1)

    in0 = (t >= C0) & (t < C1)
    in1 = t >= C1
    out = head_term \
        + jnp.where(in0, local0, 0.0) \
        + jnp.where(in1, local1, 0.0)
    out_ref[...] = out


def _pad_rows(w, n):
    return jnp.pad(w, ((0, n - w.shape[0]), (0, 0)))


@jax.jit
def kernel(input, target, W_head, W1_0, W2_0, W1_1, W2_1):
    n, d = input.shape
    rb = RB
    grid = n // rb

    x = input.astype(jnp.bfloat16)
    t3 = target.astype(jnp.int32).reshape(grid, rb, 1)
    wh = _pad_rows(W_head, HEAD_PAD).astype(jnp.bfloat16)
    w10 = W1_0.astype(jnp.bfloat16)
    w20 = _pad_rows(W2_0, 9 * CHUNK).astype(jnp.bfloat16)
    w11 = W1_1.astype(jnp.bfloat16)
    w21 = _pad_rows(W2_1, 40 * CHUNK).astype(jnp.bfloat16)

    const = lambda shape: pl.BlockSpec(shape, lambda i: (0,) * len(shape))
    out = pl.pallas_call(
        functools.partial(_fused_kernel, rb=rb),
        grid=(grid,),
        in_specs=[
            pl.BlockSpec((rb, d), lambda i: (i, 0)),
            pl.BlockSpec((1, rb, 1), lambda i: (i, 0, 0)),
            const(wh.shape),
            const(w10.shape),
            const(w20.shape),
            const(w11.shape),
            const(w21.shape),
        ],
        out_specs=pl.BlockSpec((rb, 1), lambda i: (i, 0)),
        out_shape=jax.ShapeDtypeStruct((n, 1), jnp.float32),
        compiler_params=pltpu.CompilerParams(
            dimension_semantics=("arbitrary",),
        ),
    )(x, t3, wh, w10, w20, w11, w21)

    output = out.reshape(n)
    loss = (-output).mean()
    return output, loss


# speedup vs baseline: 2.9262x; 1.0675x over previous
import functools

import jax
import jax.numpy as jnp
from jax.experimental import pallas as pl
from jax.experimental.pallas import tpu as pltpu

C0 = 2000
C1 = 20000
C2 = 100000
HEAD = 2002
HEAD_PAD = 2048
V0 = 18000
V1 = 80000
CHUNK = 2048
RB = 512


def _fused_kernel(x_ref, t_ref, wh_ref, w10_ref, w20_ref, w11_ref, w21_ref,
                  out_ref, *, rb):
    x = x_ref[...]
    t = t_ref[0]

    h0 = jax.lax.dot_general(x, w10_ref[...], (((1,), (1,)), ((), ())),
                             preferred_element_type=jnp.float32)
    h0 = h0.astype(jnp.bfloat16)
    h1 = jax.lax.dot_general(x, w11_ref[...], (((1,), (1,)), ((), ())),
                             preferred_element_type=jnp.float32)
    h1 = h1.astype(jnp.bfloat16)

    lg = jax.lax.dot_general(x, wh_ref[...], (((1,), (1,)), ((), ())),
                             preferred_element_type=jnp.float32)
    col = jax.lax.broadcasted_iota(jnp.int32, (rb, HEAD_PAD), 1)
    m = jnp.max(lg, axis=1, keepdims=True)
    s = jnp.sum(jnp.exp(lg - m), axis=1, keepdims=True)
    s = s - (HEAD_PAD - HEAD) * jnp.exp(-m)
    lse_h = m + jnp.log(s)
    gidx = jnp.where(t < C0, t, jnp.where(t < C1, C0, C0 + 1))
    at_h = jnp.sum(jnp.where(col == gidx, lg, 0.0), axis=1, keepdims=True)
    head_term = at_h - lse_h

    def tail(h, w_ref, vocab, rel):
        nchunks = w_ref.shape[0] // CHUNK

        def body(c, carry):
            m, s, at = carry
            w = w_ref[pl.ds(c * CHUNK, CHUNK), :]
            lg = jax.lax.dot_general(h, w, (((1,), (1,)), ((), ())),
                                     preferred_element_type=jnp.float32)
            col = jax.lax.broadcasted_iota(jnp.int32, (rb, CHUNK), 1) \
                + c * CHUNK
            m_new = jnp.maximum(m, jnp.max(lg, axis=1, keepdims=True))
            s = s * jnp.exp(m - m_new) \
                + jnp.sum(jnp.exp(lg - m_new), axis=1, keepdims=True)
            at = at + jnp.sum(jnp.where(col == rel, lg, 0.0), axis=1,
                              keepdims=True)
            return m_new, s, at

        neg = jnp.full((rb, 1), -jnp.inf, dtype=jnp.float32)
        zero = jnp.zeros((rb, 1), dtype=jnp.float32)
        m, s, at = jax.lax.fori_loop(0, nchunks, body, (neg, zero, zero))
        s = s - (nchunks * CHUNK - vocab) * jnp.exp(-m)
        return at - (m + jnp.log(s))

    local0 = tail(h0, w20_ref, V0, t - C0)
    local1 = tail(h1, w21_ref, V1, t - C1)

    in0 = (t >= C0) & (t < C1)
    in1 = t >= C1
    out = head_term \
        + jnp.where(in0, local0, 0.0) \
        + jnp.where(in1, local1, 0.0)
    out_ref[...] = out


def _pad_rows(w, n):
    return jnp.pad(w, ((0, n - w.shape[0]), (0, 0)))


@jax.jit
def kernel(input, target, W_head, W1_0, W2_0, W1_1, W2_1):
    n, d = input.shape
    rb = RB
    grid = n // rb

    x = input.astype(jnp.bfloat16)
    t3 = target.astype(jnp.int32).reshape(grid, rb, 1)
    wh = _pad_rows(W_head, HEAD_PAD).astype(jnp.bfloat16)
    w10 = W1_0.astype(jnp.bfloat16)
    w20 = _pad_rows(W2_0, 9 * CHUNK).astype(jnp.bfloat16)
    w11 = W1_1.astype(jnp.bfloat16)
    w21 = _pad_rows(W2_1, 40 * CHUNK).astype(jnp.bfloat16)

    const = lambda shape: pl.BlockSpec(shape, lambda i: (0,) * len(shape))
    out = pl.pallas_call(
        functools.partial(_fused_kernel, rb=rb),
        grid=(grid,),
        in_specs=[
            pl.BlockSpec((rb, d), lambda i: (i, 0)),
            pl.BlockSpec((1, rb, 1), lambda i: (i, 0, 0)),
            const(wh.shape),
            const(w10.shape),
            const(w20.shape),
            const(w11.shape),
            const(w21.shape),
        ],
        out_specs=pl.BlockSpec((rb, 1), lambda i: (i, 0)),
        out_shape=jax.ShapeDtypeStruct((n, 1), jnp.float32),
        compiler_params=pltpu.CompilerParams(
            dimension_semantics=("arbitrary",),
        ),
    )(x, t3, wh, w10, w20, w11, w21)

    output = out.reshape(n)
    loss = (-output).mean()
    return output, loss
